# Initial kernel scaffold; baseline (speedup 1.0000x reference)
import jax
import jax.numpy as jnp
from jax.experimental import pallas as pl


def kernel(x, edge_index, W_gcn, b_gcn, ln_g, ln_b, Wq, bq, Wk, bk, Wv, bv, Wo, bo):
    raise NotImplementedError("write your pallas kernel here")


# speedup vs baseline: 7.8632x; 7.8632x over previous
import functools

import jax
import jax.numpy as jnp
import numpy as np
from jax import lax
from jax.experimental import pallas as pl
from jax.experimental.pallas import tpu as pltpu
from jax.experimental.pallas import tpu_sc as plsc

N = 10000
D = 128
H = 4
DH = 32
N_PAD = 10240
NC = 2
NS = 16
NW = NC * NS
CHUNK = 128
CPW = 79
EPW = CPW * CHUNK
E_PAD = NW * EPW
RPT = N_PAD // NS
BQ = 1000
NKV = 10

_MESH = plsc.VectorSubcoreMesh(
    core_axis_name="c", subcore_axis_name="s", num_cores=NC, num_subcores=NS)


@functools.partial(
    pl.kernel,
    out_type=jax.ShapeDtypeStruct((NC, N_PAD, 16), jnp.float32),
    mesh=_MESH,
    scratch_types=[
        pltpu.VMEM((CPW, CHUNK), jnp.int32),
        pltpu.VMEM((CHUNK, 16), jnp.float32),
        pltpu.VMEM_SHARED((N_PAD, 16), jnp.float32),
    ],
)
def _sc_deg(dst3, e0rows, zrows16, out, idx_v, e0_v, acc_sh):
    c = lax.axis_index("c")
    s = lax.axis_index("s")
    wid = c * NS + s
    pltpu.sync_copy(zrows16.at[pl.ds(s * RPT, RPT)],
                    acc_sh.at[pl.ds(s * RPT, RPT)])
    pltpu.sync_copy(e0rows, e0_v)
    pltpu.sync_copy(dst3.at[wid], idx_v)
    plsc.subcore_barrier()

    def chunk(j, carry):
        pltpu.sync_copy(e0_v, acc_sh.at[idx_v.at[j]], add=True)
        return carry

    lax.fori_loop(0, CPW, chunk, 0)
    plsc.subcore_barrier()
    pltpu.sync_copy(acc_sh.at[pl.ds(s * RPT, RPT)],
                    out.at[c, pl.ds(s * RPT, RPT)])


@functools.partial(
    pl.kernel,
    out_type=jax.ShapeDtypeStruct((NC, N_PAD, D), jnp.float32),
    mesh=_MESH,
    scratch_types=[
        pltpu.VMEM((CPW, CHUNK), jnp.int32),
        pltpu.VMEM((CPW, CHUNK), jnp.int32),
        pltpu.VMEM((CHUNK, D), jnp.float32),
        pltpu.VMEM_SHARED((N_PAD, D), jnp.float32),
        pltpu.SemaphoreType.DMA,
    ],
)
def _sc_msg(g_hbm, src3, dst3, zrows, out, sidx_v, didx_v, rows_v, acc_sh, sem):
    c = lax.axis_index("c")
    s = lax.axis_index("s")
    wid = c * NS + s
    pltpu.sync_copy(zrows.at[pl.ds(s * RPT, RPT)],
                    acc_sh.at[pl.ds(s * RPT, RPT)])
    pltpu.sync_copy(src3.at[wid], sidx_v)
    pltpu.sync_copy(dst3.at[wid], didx_v)
    plsc.subcore_barrier()

    def chunk(j, carry):
        pltpu.async_copy(g_hbm.at[sidx_v.at[j]], rows_v, sem).wait()
        pltpu.sync_copy(rows_v, acc_sh.at[didx_v.at[j]], add=True)
        return carry

    lax.fori_loop(0, CPW, chunk, 0)
    plsc.subcore_barrier()
    pltpu.sync_copy(acc_sh.at[pl.ds(s * RPT, RPT)],
                    out.at[c, pl.ds(s * RPT, RPT)])


def _prep_body(x_ref, w_ref, deg_ref, g_ref, hd2_ref):
    dcnt = deg_ref[0, :, 0:1] + deg_ref[1, :, 0:1] + 1.0
    dinv = lax.rsqrt(dcnt)
    h = jnp.dot(x_ref[...], w_ref[...], preferred_element_type=jnp.float32)
    g_ref[...] = h * dinv
    hd2_ref[...] = h * (dinv * dinv)


def _tc_prep(x_pad, w_gcn, deg2):
    blk = N_PAD // NKV
    return pl.pallas_call(
        _prep_body,
        grid=(NKV,),
        in_specs=[
            pl.BlockSpec((blk, D), lambda i: (i, 0)),
            pl.BlockSpec((D, D), lambda i: (0, 0)),
            pl.BlockSpec((NC, blk, 16), lambda i: (0, i, 0)),
        ],
        out_specs=[
            pl.BlockSpec((blk, D), lambda i: (i, 0)),
            pl.BlockSpec((blk, D), lambda i: (i, 0)),
        ],
        out_shape=[
            jax.ShapeDtypeStruct((N_PAD, D), jnp.float32),
            jax.ShapeDtypeStruct((N_PAD, D), jnp.float32),
        ],
    )(x_pad, w_gcn, deg2)


def _mid_body(s_ref, deg_ref, hd2_ref, bg_ref, lg_ref, lb_ref,
              wq_ref, bq_ref, wk_ref, bk_ref, wv_ref, bv_ref,
              hn_ref, q_ref, k_ref, v_ref):
    dcnt = deg_ref[0, :, 0:1] + deg_ref[1, :, 0:1] + 1.0
    dinv = lax.rsqrt(dcnt)
    agg = dinv * (s_ref[0] + s_ref[1]) + hd2_ref[...] + bg_ref[...]
    mu = jnp.mean(agg, axis=-1, keepdims=True)
    var = jnp.mean((agg - mu) ** 2, axis=-1, keepdims=True)
    hn = (agg - mu) * lax.rsqrt(var + 1e-5) * lg_ref[...] + lb_ref[...]
    hn_ref[...] = hn
    q_ref[...] = jnp.dot(hn, wq_ref[...],
                         preferred_element_type=jnp.float32) + bq_ref[...]
    k_ref[...] = jnp.dot(hn, wk_ref[...],
                         preferred_element_type=jnp.float32) + bk_ref[...]
    v_ref[...] = jnp.dot(hn, wv_ref[...],
                         preferred_element_type=jnp.float32) + bv_ref[...]


def _tc_mid(s, deg2, hd2, bg, lg, lb, wq, bq, wk, bk, wv, bv):
    row_spec = pl.BlockSpec((BQ, D), lambda i: (i, 0))
    w_spec = pl.BlockSpec((D, D), lambda i: (0, 0))
    b_spec = pl.BlockSpec((1, D), lambda i: (0, 0))
    return pl.pallas_call(
        _mid_body,
        grid=(NKV,),
        in_specs=[
            pl.BlockSpec((NC, BQ, D), lambda i: (0, i, 0)),
            pl.BlockSpec((NC, BQ, 16), lambda i: (0, i, 0)),
            row_spec, b_spec, b_spec, b_spec,
            w_spec, b_spec, w_spec, b_spec, w_spec, b_spec,
        ],
        out_specs=[row_spec, row_spec, row_spec, row_spec],
        out_shape=[jax.ShapeDtypeStruct((N, D), jnp.float32)] * 4,
    )(s, deg2, hd2, bg, lg, lb, wq, bq, wk, bk, wv, bv)


def _attn_body(q_ref, k_ref, v_ref, hn_ref, wo_ref, bo_ref, o_ref):
    scale = 1.0 / np.sqrt(DH)
    ctxs = []
    for hh in range(H):
        qh = q_ref[:, hh * DH:(hh + 1) * DH] * scale

        def kv_step(j, carry):
            m, l, acc = carry
            kh = k_ref[pl.ds(j * BQ, BQ), hh * DH:(hh + 1) * DH]
            vh = v_ref[pl.ds(j * BQ, BQ), hh * DH:(hh + 1) * DH]
            sj = lax.dot_general(qh, kh, (((1,), (1,)), ((), ())),
                                 preferred_element_type=jnp.float32)
            mj = jnp.maximum(m, jnp.max(sj, axis=-1, keepdims=True))
            p = jnp.exp(sj - mj)
            alpha = jnp.exp(m - mj)
            l = l * alpha + jnp.sum(p, axis=-1, keepdims=True)
            acc = acc * alpha + jnp.dot(p, vh,
                                        preferred_element_type=jnp.float32)
            return mj, l, acc

        m0 = jnp.full((BQ, 1), -jnp.inf, jnp.float32)
        l0 = jnp.zeros((BQ, 1), jnp.float32)
        a0 = jnp.zeros((BQ, DH), jnp.float32)
        m, l, acc = lax.fori_loop(0, NKV, kv_step, (m0, l0, a0))
        ctxs.append(acc / l)
    ctx = jnp.concatenate(ctxs, axis=1)
    o_ref[...] = hn_ref[...] + jnp.dot(
        ctx, wo_ref[...], preferred_element_type=jnp.float32) + bo_ref[...]


def _tc_attn(q, k, v, hn, wo, bo):
    row_spec = pl.BlockSpec((BQ, D), lambda i: (i, 0))
    full_spec = pl.BlockSpec((N, D), lambda i: (0, 0))
    return pl.pallas_call(
        _attn_body,
        grid=(NKV,),
        in_specs=[row_spec, full_spec, full_spec, row_spec,
                  pl.BlockSpec((D, D), lambda i: (0, 0)),
                  pl.BlockSpec((1, D), lambda i: (0, 0))],
        out_specs=row_spec,
        out_shape=jax.ShapeDtypeStruct((N, D), jnp.float32),
    )(q, k, v, hn, wo, bo)


def kernel(x, edge_index, W_gcn, b_gcn, ln_g, ln_b,
           Wq, bq, Wk, bk, Wv, bv, Wo, bo):
    src = edge_index[0].astype(jnp.int32)
    dst = edge_index[1].astype(jnp.int32)
    pad = jnp.full((E_PAD - src.shape[0],), N, jnp.int32)
    src3 = jnp.concatenate([src, pad]).reshape(NW, CPW, CHUNK)
    dst3 = jnp.concatenate([dst, pad]).reshape(NW, CPW, CHUNK)

    e0rows = jnp.zeros((CHUNK, 16), jnp.float32).at[:, 0].set(1.0)
    zrows16 = jnp.zeros((N_PAD, 16), jnp.float32)
    zrows = jnp.zeros((N_PAD, D), jnp.float32)
    x_pad = jnp.concatenate(
        [x, jnp.zeros((N_PAD - N, D), jnp.float32)], axis=0)

    deg2 = _sc_deg(dst3, e0rows, zrows16)
    g, hd2 = _tc_prep(x_pad, W_gcn, deg2)
    s = _sc_msg(g, src3, dst3, zrows)
    hn, q, k, v = _tc_mid(
        s, deg2, hd2,
        b_gcn.reshape(1, D), ln_g.reshape(1, D), ln_b.reshape(1, D),
        Wq, bq.reshape(1, D), Wk, bk.reshape(1, D), Wv, bv.reshape(1, D))
    return _tc_attn(q, k, v, hn, Wo, bo.reshape(1, D))
